# baseline (speedup 1.0000x reference)
import functools

import jax
import jax.numpy as jnp
from jax import lax
from jax.experimental import pallas as pl
from jax.experimental.pallas import tpu as pltpu
from jax.experimental.pallas import tpu_sc as plsc

_MAX_ITEMS = 1000000
_B = 16384
_D = 128
_L = 16
_NW = 32
_WEIGHT = 0.1
_MOMENT = 0.5
_WARM_UP = 1.0 / 1000.0
_COOL_DOWN = 1.0 / 100000.0


def _sc_call(activations, samples, hist_ref):
    mesh = plsc.VectorSubcoreMesh(core_axis_name="c", subcore_axis_name="s")

    @functools.partial(
        pl.kernel,
        out_type=(
            jax.ShapeDtypeStruct((_NW, _L), jnp.float32),
            jax.ShapeDtypeStruct((_B + _L, _D), jnp.float32),
        ),
        mesh=mesh,
        compiler_params=pltpu.CompilerParams(needs_layout_passes=False),
        scratch_types=[
            pltpu.VMEM((_B,), jnp.int32),
            pltpu.VMEM((_B,), jnp.int32),
            pltpu.VMEM((_L, _D), jnp.float32),
            pltpu.VMEM((_L, _D), jnp.float32),
            pltpu.VMEM((_L, _D), jnp.float32),
            pltpu.VMEM((_L, _D), jnp.float32),
            pltpu.VMEM((_L,), jnp.float32),
            pltpu.SemaphoreType.DMA,
            pltpu.SemaphoreType.DMA,
        ],
    )
    def body(act_hbm, smp_hbm, hist, loss_hbm, upd_hbm,
             smp_v, pos_v, old_v, act_v, u_v, cur_v, lss_v, s0, s1):
        wid = lax.axis_index("s") * 2 + lax.axis_index("c")
        lanes = lax.iota(jnp.int32, _L)

        pltpu.sync_copy(smp_hbm, smp_v)

        def build(v, cnt):
            s = smp_v[pl.ds(v * _L, _L)]
            own = (jnp.bitwise_and(s, _NW - 1) == wid) & (s > 0) & (s < _MAX_ITEMS)
            inc = jnp.cumsum(own.astype(jnp.int32))
            plsc.store_scatter(pos_v, [cnt + inc - 1], v * _L + lanes, mask=own)
            return cnt + jnp.max(inc)

        n_own = lax.fori_loop(0, _B // _L, build, jnp.int32(0))
        nchunks = (n_own + _L - 1) // _L

        def load_chunk(k):
            base = k * _L
            valid = (base + lanes) < n_own
            pos = jnp.where(valid, pos_v[pl.ds(base, _L)], 0)
            sid = plsc.load_gather(smp_v, [pos])
            posu = jnp.where(valid, pos, _B + lanes)
            return valid, pos, sid, posu

        def p1(k, lossvec):
            valid, pos, sid, posu = load_chunk(k)
            sid_dma = jnp.where(valid, sid, _MAX_ITEMS)
            cp_a = pltpu.async_copy(act_hbm.at[pos], act_v, s0)
            cp_h = pltpu.async_copy(hist.at[sid_dma], old_v, s1)
            cp_a.wait()
            cp_h.wait()

            def col(c, lv):
                colv = lanes * 0 + c
                o = plsc.load_gather(old_v, [lanes, colv])
                a = plsc.load_gather(act_v, [lanes, colv])
                d = jnp.where(valid, o - a, 0.0)
                plsc.store_scatter(u_v, [lanes, colv], (_MOMENT - 1.0) * d)
                return lv + d * d

            lossvec = lax.fori_loop(0, _D, col, lossvec)
            pltpu.async_copy(u_v, upd_hbm.at[posu], s0).wait()
            return lossvec

        lossvec = lax.fori_loop(0, nchunks, p1, jnp.zeros((_L,), jnp.float32))
        lss_v[...] = lossvec
        pltpu.sync_copy(lss_v, loss_hbm.at[wid])

        def p2(k, carry):
            valid, pos, sid, posu = load_chunk(k)
            sidr = jnp.where(valid, sid, _MAX_ITEMS + lanes)
            rank = jnp.zeros((_L,), jnp.int32)
            for j in range(_L - 1):
                sj = jnp.sum(jnp.where(lanes == j, sidr, 0))
                rank = rank + jnp.where((lanes > j) & (sidr == sj), 1, 0)
            maxrank = jnp.max(rank)
            pltpu.async_copy(upd_hbm.at[posu], u_v, s0).wait()

            def round_body(r):
                sel = (rank == r) & valid
                idx = jnp.where(sel, sid, _MAX_ITEMS)
                pltpu.async_copy(hist.at[idx], cur_v, s1).wait()

                def addc(c, t):
                    colv = lanes * 0 + c
                    cu = plsc.load_gather(cur_v, [lanes, colv])
                    uu = plsc.load_gather(u_v, [lanes, colv])
                    plsc.store_scatter(
                        cur_v, [lanes, colv], cu + jnp.where(sel, uu, 0.0))
                    return t

                lax.fori_loop(0, _D, addc, jnp.int32(0))
                pltpu.async_copy(cur_v, hist.at[idx], s1).wait()
                return r + 1

            lax.while_loop(lambda r: r <= maxrank, round_body, jnp.int32(0))
            return carry

        lax.fori_loop(0, nchunks, p2, jnp.int32(0))

    return body(activations, samples, hist_ref)


def kernel(activations, samples, history, iterations):
    warm_up = _WARM_UP * iterations
    cool_down = _COOL_DOWN * iterations
    weight = _WEIGHT * warm_up / (1.0 + warm_up) / (1.0 + cool_down)

    hist_ref = jax.new_ref(history)
    loss_parts, _ = _sc_call(activations, samples, hist_ref)
    new_history = hist_ref[...]
    reg_loss = jnp.sum(loss_parts) * (weight / (_B * float(_D)))
    return activations, reg_loss, new_history, iterations + 1.0
